# Initial kernel scaffold; baseline (speedup 1.0000x reference)
import jax
import jax.numpy as jnp
from jax.experimental import pallas as pl


def kernel(node_inds, adj_mat_inds, init_hydrogens, init_charge, init_is_in_ring, init_is_aromatic, init_chirality, n_emb, e_emb, h_emb, charge_emb, ring_emb, arom_emb, chir_emb):
    raise NotImplementedError("write your pallas kernel here")


# speedup vs baseline: 1.6331x; 1.6331x over previous
import jax
import jax.numpy as jnp
from jax.experimental import pallas as pl
from jax.experimental.pallas import tpu as pltpu

B = 1024
N = 32
DIM_H = 128
DIM_K = 2
NUM_NODE_CLASSES = 42
NUM_EDGE_CLASSES = 6
NUM_H_CLASSES = 6
NUM_CHARGE_CLASSES = 6
NUM_RING_CLASSES = 3
NUM_AROM_CLASSES = 3
NUM_CHIR_CLASSES = 5

_NODE_SIZES = (NUM_NODE_CLASSES, NUM_H_CLASSES, NUM_CHARGE_CLASSES,
               NUM_RING_CLASSES, NUM_AROM_CLASSES, NUM_CHIR_CLASSES)
_NODE_OFFSETS = tuple(sum(_NODE_SIZES[:i]) for i in range(len(_NODE_SIZES)))

GRID = 512
RE = (B * N * N) // GRID
RN = (B * N) // GRID


def _fused_body(eidx_ref, nidx_ref, e_tab_ref, n_tab_ref,
                edge_out_ref, node_out_ref):
    eidx = eidx_ref[0, 0, :]
    e_iota = jax.lax.broadcasted_iota(jnp.int32, (RE, 8), 1)
    e_onehot = (eidx[:, None] == e_iota).astype(jnp.float32)
    edge_out_ref[...] = jnp.dot(e_onehot, e_tab_ref[...],
                                preferred_element_type=jnp.float32)

    n_iota = jax.lax.broadcasted_iota(jnp.int32, (RN, 128), 1)
    m = jnp.zeros((RN, 128), jnp.float32)
    for t, off in enumerate(_NODE_OFFSETS):
        idx_t = nidx_ref[0, t, :]
        m = m + (idx_t[:, None] + off == n_iota).astype(jnp.float32)
    node_out_ref[...] = jnp.dot(m, n_tab_ref[...],
                                preferred_element_type=jnp.float32)


def kernel(node_inds, adj_mat_inds, init_hydrogens, init_charge,
           init_is_in_ring, init_is_aromatic, init_chirality,
           n_emb, e_emb, h_emb, charge_emb, ring_emb, arom_emb, chir_emb):
    eidx = adj_mat_inds.reshape(GRID, 1, RE)
    nidx = jnp.stack([a.reshape(GRID, RN) for a in
                      (node_inds, init_hydrogens, init_charge,
                       init_is_in_ring, init_is_aromatic, init_chirality)],
                     axis=1)

    e_tab = jnp.zeros((8, DIM_H * DIM_K), jnp.float32).at[:NUM_EDGE_CLASSES].set(e_emb)
    n_tab = jnp.zeros((128, DIM_H), jnp.float32)
    for tab, off in zip((n_emb, h_emb, charge_emb, ring_emb, arom_emb, chir_emb),
                        _NODE_OFFSETS):
        n_tab = n_tab.at[off:off + tab.shape[0]].set(tab)

    edge_out, node_out = pl.pallas_call(
        _fused_body,
        grid=(GRID,),
        in_specs=[
            pl.BlockSpec((1, 1, RE), lambda i: (i, 0, 0)),
            pl.BlockSpec((1, 6, RN), lambda i: (i, 0, 0)),
            pl.BlockSpec((8, DIM_H * DIM_K), lambda i: (0, 0)),
            pl.BlockSpec((128, DIM_H), lambda i: (0, 0)),
        ],
        out_specs=[
            pl.BlockSpec((RE, DIM_H * DIM_K), lambda i: (i, 0)),
            pl.BlockSpec((RN, DIM_H), lambda i: (i, 0)),
        ],
        out_shape=[
            jax.ShapeDtypeStruct((B * N * N, DIM_H * DIM_K), jnp.float32),
            jax.ShapeDtypeStruct((B * N, DIM_H), jnp.float32),
        ],
    )(eidx, nidx, e_tab, n_tab)

    return (node_out.reshape(B, N, DIM_H),
            edge_out.reshape(B, N, N, DIM_H, DIM_K))


# speedup vs baseline: 23.0798x; 14.1325x over previous
import jax
import jax.numpy as jnp
from jax.experimental import pallas as pl
from jax.experimental.pallas import tpu as pltpu

B = 1024
N = 32
DIM_H = 128
DIM_K = 2
NUM_NODE_CLASSES = 42
NUM_EDGE_CLASSES = 6
NUM_H_CLASSES = 6
NUM_CHARGE_CLASSES = 6
NUM_RING_CLASSES = 3
NUM_AROM_CLASSES = 3
NUM_CHIR_CLASSES = 5

_NODE_SIZES = (NUM_NODE_CLASSES, NUM_H_CLASSES, NUM_CHARGE_CLASSES,
               NUM_RING_CLASSES, NUM_AROM_CLASSES, NUM_CHIR_CLASSES)
_NODE_OFFSETS = tuple(sum(_NODE_SIZES[:i]) for i in range(len(_NODE_SIZES)))

GRID = 512
RE = (B * N * N * DIM_K) // GRID
RN = (B * N) // GRID


def _fused_body(eidx_ref, nidx_ref, e_tab_ref, n_tab_ref,
                edge_out_ref, node_out_ref):
    eidx = eidx_ref[0, 0, :]
    e_iota = jax.lax.broadcasted_iota(jnp.int32, (RE, 16), 1)
    e_onehot = (eidx[:, None] == e_iota).astype(jnp.float32)
    edge_out_ref[...] = jnp.dot(e_onehot, e_tab_ref[...],
                                preferred_element_type=jnp.float32)

    n_iota = jax.lax.broadcasted_iota(jnp.int32, (RN, 128), 1)
    m = jnp.zeros((RN, 128), jnp.float32)
    for t, off in enumerate(_NODE_OFFSETS):
        idx_t = nidx_ref[0, t, :]
        m = m + (idx_t[:, None] + off == n_iota).astype(jnp.float32)
    node_out_ref[...] = jnp.dot(m, n_tab_ref[...],
                                preferred_element_type=jnp.float32)


def kernel(node_inds, adj_mat_inds, init_hydrogens, init_charge,
           init_is_in_ring, init_is_aromatic, init_chirality,
           n_emb, e_emb, h_emb, charge_emb, ring_emb, arom_emb, chir_emb):
    eidx2 = (adj_mat_inds.reshape(-1, 1) * 2
             + jnp.arange(2, dtype=adj_mat_inds.dtype))
    eidx2 = eidx2.reshape(GRID, 1, RE)
    nidx = jnp.stack([a.reshape(GRID, RN) for a in
                      (node_inds, init_hydrogens, init_charge,
                       init_is_in_ring, init_is_aromatic, init_chirality)],
                     axis=1)

    e_tab = jnp.zeros((16, DIM_H), jnp.float32).at[:2 * NUM_EDGE_CLASSES].set(
        e_emb.reshape(NUM_EDGE_CLASSES, DIM_H, DIM_K)
             .transpose(0, 2, 1).reshape(2 * NUM_EDGE_CLASSES, DIM_H))
    n_tab = jnp.zeros((128, DIM_H), jnp.float32)
    for tab, off in zip((n_emb, h_emb, charge_emb, ring_emb, arom_emb, chir_emb),
                        _NODE_OFFSETS):
        n_tab = n_tab.at[off:off + tab.shape[0]].set(tab)

    edge_out, node_out = pl.pallas_call(
        _fused_body,
        grid=(GRID,),
        in_specs=[
            pl.BlockSpec((1, 1, RE), lambda i: (i, 0, 0)),
            pl.BlockSpec((1, 6, RN), lambda i: (i, 0, 0)),
            pl.BlockSpec((16, DIM_H), lambda i: (0, 0)),
            pl.BlockSpec((128, DIM_H), lambda i: (0, 0)),
        ],
        out_specs=[
            pl.BlockSpec((RE, DIM_H), lambda i: (i, 0)),
            pl.BlockSpec((RN, DIM_H), lambda i: (i, 0)),
        ],
        out_shape=[
            jax.ShapeDtypeStruct((B * N * N * DIM_K, DIM_H), jnp.float32),
            jax.ShapeDtypeStruct((B * N, DIM_H), jnp.float32),
        ],
    )(eidx2, nidx, e_tab, n_tab)

    edge5 = edge_out.reshape(B, N, N, DIM_K, DIM_H).swapaxes(-1, -2)
    return (node_out.reshape(B, N, DIM_H), edge5)


# speedup vs baseline: 29.2225x; 1.2662x over previous
import jax
import jax.numpy as jnp
from jax.experimental import pallas as pl
from jax.experimental.pallas import tpu as pltpu

B = 1024
N = 32
DIM_H = 128
DIM_K = 2
NUM_NODE_CLASSES = 42
NUM_EDGE_CLASSES = 6
NUM_H_CLASSES = 6
NUM_CHARGE_CLASSES = 6
NUM_RING_CLASSES = 3
NUM_AROM_CLASSES = 3
NUM_CHIR_CLASSES = 5

_NODE_SIZES = (NUM_NODE_CLASSES, NUM_H_CLASSES, NUM_CHARGE_CLASSES,
               NUM_RING_CLASSES, NUM_AROM_CLASSES, NUM_CHIR_CLASSES)
_NODE_OFFSETS = tuple(sum(_NODE_SIZES[:i]) for i in range(len(_NODE_SIZES)))

GRID = 256
RE = (B * N * N * DIM_K) // GRID
RN = (B * N) // GRID


def _fused_body(eidx_ref, nidx_ref, e_tab_ref, n_tab_ref,
                edge_out_ref, node_out_ref):
    eidx = eidx_ref[0, 0, :]
    e_iota = jax.lax.broadcasted_iota(jnp.int32, (RE, 16), 1)
    e_onehot = (eidx[:, None] == e_iota).astype(jnp.float32)
    edge_out_ref[...] = jnp.dot(e_onehot, e_tab_ref[...],
                                preferred_element_type=jnp.float32)

    n_iota = jax.lax.broadcasted_iota(jnp.int32, (RN, 128), 1)
    m = jnp.zeros((RN, 128), jnp.float32)
    for t, off in enumerate(_NODE_OFFSETS):
        idx_t = nidx_ref[0, t, :]
        m = m + (idx_t[:, None] + off == n_iota).astype(jnp.float32)
    node_out_ref[...] = jnp.dot(m, n_tab_ref[...],
                                preferred_element_type=jnp.float32)


def kernel(node_inds, adj_mat_inds, init_hydrogens, init_charge,
           init_is_in_ring, init_is_aromatic, init_chirality,
           n_emb, e_emb, h_emb, charge_emb, ring_emb, arom_emb, chir_emb):
    eidx2 = (adj_mat_inds.reshape(-1, 1) * 2
             + jnp.arange(2, dtype=adj_mat_inds.dtype))
    eidx2 = eidx2.reshape(GRID, 1, RE)
    nidx = jnp.stack([a.reshape(GRID, RN) for a in
                      (node_inds, init_hydrogens, init_charge,
                       init_is_in_ring, init_is_aromatic, init_chirality)],
                     axis=1)

    e_tab = jnp.zeros((16, DIM_H), jnp.float32).at[:2 * NUM_EDGE_CLASSES].set(
        e_emb.reshape(NUM_EDGE_CLASSES, DIM_H, DIM_K)
             .transpose(0, 2, 1).reshape(2 * NUM_EDGE_CLASSES, DIM_H))
    n_tab = jnp.zeros((128, DIM_H), jnp.float32)
    for tab, off in zip((n_emb, h_emb, charge_emb, ring_emb, arom_emb, chir_emb),
                        _NODE_OFFSETS):
        n_tab = n_tab.at[off:off + tab.shape[0]].set(tab)

    edge_out, node_out = pl.pallas_call(
        _fused_body,
        grid=(GRID,),
        in_specs=[
            pl.BlockSpec((1, 1, RE), lambda i: (i, 0, 0)),
            pl.BlockSpec((1, 6, RN), lambda i: (i, 0, 0)),
            pl.BlockSpec((16, DIM_H), lambda i: (0, 0)),
            pl.BlockSpec((128, DIM_H), lambda i: (0, 0)),
        ],
        out_specs=[
            pl.BlockSpec((RE, DIM_H), lambda i: (i, 0)),
            pl.BlockSpec((RN, DIM_H), lambda i: (i, 0)),
        ],
        out_shape=[
            jax.ShapeDtypeStruct((B * N * N * DIM_K, DIM_H), jnp.float32),
            jax.ShapeDtypeStruct((B * N, DIM_H), jnp.float32),
        ],
    )(eidx2, nidx, e_tab, n_tab)

    edge5 = edge_out.reshape(B, N, N, DIM_K, DIM_H).swapaxes(-1, -2)
    return (node_out.reshape(B, N, DIM_H), edge5)


# speedup vs baseline: 34.1818x; 1.1697x over previous
import jax
import jax.numpy as jnp
from jax.experimental import pallas as pl
from jax.experimental.pallas import tpu as pltpu

B = 1024
N = 32
DIM_H = 128
DIM_K = 2
NUM_NODE_CLASSES = 42
NUM_EDGE_CLASSES = 6
NUM_H_CLASSES = 6
NUM_CHARGE_CLASSES = 6
NUM_RING_CLASSES = 3
NUM_AROM_CLASSES = 3
NUM_CHIR_CLASSES = 5

_NODE_SIZES = (NUM_NODE_CLASSES, NUM_H_CLASSES, NUM_CHARGE_CLASSES,
               NUM_RING_CLASSES, NUM_AROM_CLASSES, NUM_CHIR_CLASSES)
_NODE_OFFSETS = tuple(sum(_NODE_SIZES[:i]) for i in range(len(_NODE_SIZES)))

GRID = 128
RE = (B * N * N * DIM_K) // GRID
RN = (B * N) // GRID


def _fused_body(eidx_ref, nidx_ref, e_tab_ref, n_tab_ref,
                edge_out_ref, node_out_ref):
    eidx = eidx_ref[0, 0, :]
    e_iota = jax.lax.broadcasted_iota(jnp.int32, (RE, 16), 1)
    e_onehot = (eidx[:, None] == e_iota).astype(jnp.float32)
    edge_out_ref[...] = jnp.dot(e_onehot, e_tab_ref[...],
                                preferred_element_type=jnp.float32)

    n_iota = jax.lax.broadcasted_iota(jnp.int32, (RN, 128), 1)
    m = jnp.zeros((RN, 128), jnp.float32)
    for t, off in enumerate(_NODE_OFFSETS):
        idx_t = nidx_ref[0, t, :]
        m = m + (idx_t[:, None] + off == n_iota).astype(jnp.float32)
    node_out_ref[...] = jnp.dot(m, n_tab_ref[...],
                                preferred_element_type=jnp.float32)


def kernel(node_inds, adj_mat_inds, init_hydrogens, init_charge,
           init_is_in_ring, init_is_aromatic, init_chirality,
           n_emb, e_emb, h_emb, charge_emb, ring_emb, arom_emb, chir_emb):
    eidx2 = (adj_mat_inds.reshape(-1, 1) * 2
             + jnp.arange(2, dtype=adj_mat_inds.dtype))
    eidx2 = eidx2.reshape(GRID, 1, RE)
    nidx = jnp.stack([a.reshape(GRID, RN) for a in
                      (node_inds, init_hydrogens, init_charge,
                       init_is_in_ring, init_is_aromatic, init_chirality)],
                     axis=1)

    e_tab = jnp.zeros((16, DIM_H), jnp.float32).at[:2 * NUM_EDGE_CLASSES].set(
        e_emb.reshape(NUM_EDGE_CLASSES, DIM_H, DIM_K)
             .transpose(0, 2, 1).reshape(2 * NUM_EDGE_CLASSES, DIM_H))
    n_tab = jnp.zeros((128, DIM_H), jnp.float32)
    for tab, off in zip((n_emb, h_emb, charge_emb, ring_emb, arom_emb, chir_emb),
                        _NODE_OFFSETS):
        n_tab = n_tab.at[off:off + tab.shape[0]].set(tab)

    edge_out, node_out = pl.pallas_call(
        _fused_body,
        grid=(GRID,),
        in_specs=[
            pl.BlockSpec((1, 1, RE), lambda i: (i, 0, 0)),
            pl.BlockSpec((1, 6, RN), lambda i: (i, 0, 0)),
            pl.BlockSpec((16, DIM_H), lambda i: (0, 0)),
            pl.BlockSpec((128, DIM_H), lambda i: (0, 0)),
        ],
        out_specs=[
            pl.BlockSpec((RE, DIM_H), lambda i: (i, 0)),
            pl.BlockSpec((RN, DIM_H), lambda i: (i, 0)),
        ],
        out_shape=[
            jax.ShapeDtypeStruct((B * N * N * DIM_K, DIM_H), jnp.float32),
            jax.ShapeDtypeStruct((B * N, DIM_H), jnp.float32),
        ],
    )(eidx2, nidx, e_tab, n_tab)

    edge5 = edge_out.reshape(B, N, N, DIM_K, DIM_H).swapaxes(-1, -2)
    return (node_out.reshape(B, N, DIM_H), edge5)


# speedup vs baseline: 34.6096x; 1.0125x over previous
import jax
import jax.numpy as jnp
from jax.experimental import pallas as pl
from jax.experimental.pallas import tpu as pltpu

B = 1024
N = 32
DIM_H = 128
DIM_K = 2
NUM_NODE_CLASSES = 42
NUM_EDGE_CLASSES = 6
NUM_H_CLASSES = 6
NUM_CHARGE_CLASSES = 6
NUM_RING_CLASSES = 3
NUM_AROM_CLASSES = 3
NUM_CHIR_CLASSES = 5

_NODE_SIZES = (NUM_NODE_CLASSES, NUM_H_CLASSES, NUM_CHARGE_CLASSES,
               NUM_RING_CLASSES, NUM_AROM_CLASSES, NUM_CHIR_CLASSES)
_NODE_OFFSETS = tuple(sum(_NODE_SIZES[:i]) for i in range(len(_NODE_SIZES)))

GRID = 64
RE = (B * N * N * DIM_K) // GRID
RN = (B * N) // GRID


def _fused_body(eidx_ref, nidx_ref, e_tab_ref, n_tab_ref,
                edge_out_ref, node_out_ref):
    eidx = eidx_ref[0, 0, :]
    e_iota = jax.lax.broadcasted_iota(jnp.int32, (RE, 16), 1)
    e_onehot = (eidx[:, None] == e_iota).astype(jnp.float32)
    edge_out_ref[...] = jnp.dot(e_onehot, e_tab_ref[...],
                                preferred_element_type=jnp.float32)

    n_iota = jax.lax.broadcasted_iota(jnp.int32, (RN, 128), 1)
    m = jnp.zeros((RN, 128), jnp.float32)
    for t, off in enumerate(_NODE_OFFSETS):
        idx_t = nidx_ref[0, t, :]
        m = m + (idx_t[:, None] + off == n_iota).astype(jnp.float32)
    node_out_ref[...] = jnp.dot(m, n_tab_ref[...],
                                preferred_element_type=jnp.float32)


def kernel(node_inds, adj_mat_inds, init_hydrogens, init_charge,
           init_is_in_ring, init_is_aromatic, init_chirality,
           n_emb, e_emb, h_emb, charge_emb, ring_emb, arom_emb, chir_emb):
    eidx2 = (adj_mat_inds.reshape(-1, 1) * 2
             + jnp.arange(2, dtype=adj_mat_inds.dtype))
    eidx2 = eidx2.reshape(GRID, 1, RE)
    nidx = jnp.stack([a.reshape(GRID, RN) for a in
                      (node_inds, init_hydrogens, init_charge,
                       init_is_in_ring, init_is_aromatic, init_chirality)],
                     axis=1)

    e_tab = jnp.zeros((16, DIM_H), jnp.float32).at[:2 * NUM_EDGE_CLASSES].set(
        e_emb.reshape(NUM_EDGE_CLASSES, DIM_H, DIM_K)
             .transpose(0, 2, 1).reshape(2 * NUM_EDGE_CLASSES, DIM_H))
    n_tab = jnp.zeros((128, DIM_H), jnp.float32)
    for tab, off in zip((n_emb, h_emb, charge_emb, ring_emb, arom_emb, chir_emb),
                        _NODE_OFFSETS):
        n_tab = n_tab.at[off:off + tab.shape[0]].set(tab)

    edge_out, node_out = pl.pallas_call(
        _fused_body,
        grid=(GRID,),
        in_specs=[
            pl.BlockSpec((1, 1, RE), lambda i: (i, 0, 0)),
            pl.BlockSpec((1, 6, RN), lambda i: (i, 0, 0)),
            pl.BlockSpec((16, DIM_H), lambda i: (0, 0)),
            pl.BlockSpec((128, DIM_H), lambda i: (0, 0)),
        ],
        out_specs=[
            pl.BlockSpec((RE, DIM_H), lambda i: (i, 0)),
            pl.BlockSpec((RN, DIM_H), lambda i: (i, 0)),
        ],
        out_shape=[
            jax.ShapeDtypeStruct((B * N * N * DIM_K, DIM_H), jnp.float32),
            jax.ShapeDtypeStruct((B * N, DIM_H), jnp.float32),
        ],
    )(eidx2, nidx, e_tab, n_tab)

    edge5 = edge_out.reshape(B, N, N, DIM_K, DIM_H).swapaxes(-1, -2)
    return (node_out.reshape(B, N, DIM_H), edge5)


# speedup vs baseline: 36.1301x; 1.0439x over previous
import jax
import jax.numpy as jnp
from jax.experimental import pallas as pl
from jax.experimental.pallas import tpu as pltpu

B = 1024
N = 32
DIM_H = 128
DIM_K = 2
NUM_NODE_CLASSES = 42
NUM_EDGE_CLASSES = 6
NUM_H_CLASSES = 6
NUM_CHARGE_CLASSES = 6
NUM_RING_CLASSES = 3
NUM_AROM_CLASSES = 3
NUM_CHIR_CLASSES = 5

_NODE_SIZES = (NUM_NODE_CLASSES, NUM_H_CLASSES, NUM_CHARGE_CLASSES,
               NUM_RING_CLASSES, NUM_AROM_CLASSES, NUM_CHIR_CLASSES)
_NODE_OFFSETS = tuple(sum(_NODE_SIZES[:i]) for i in range(len(_NODE_SIZES)))

GRID = 64
RE = (B * N * N * DIM_K) // GRID
RN = (B * N) // GRID


def _fused_body(eidx_ref, nidx_ref, e_tab_ref, n_tab_ref,
                edge_out_ref, node_out_ref):
    eidx = jnp.broadcast_to(eidx_ref[0, :, :], (16, RE))
    e_iota = jax.lax.broadcasted_iota(jnp.int32, (16, RE), 0)
    e_onehot_t = (eidx == e_iota).astype(jnp.float32)
    edge_out_ref[...] = jax.lax.dot_general(
        e_onehot_t, e_tab_ref[...], (((0,), (0,)), ((), ())),
        preferred_element_type=jnp.float32)

    n_iota = jax.lax.broadcasted_iota(jnp.int32, (RN, 128), 1)
    m = jnp.zeros((RN, 128), jnp.float32)
    for t, off in enumerate(_NODE_OFFSETS):
        idx_t = nidx_ref[0, t, :]
        m = m + (idx_t[:, None] + off == n_iota).astype(jnp.float32)
    node_out_ref[...] = jnp.dot(m, n_tab_ref[...],
                                preferred_element_type=jnp.float32)


def kernel(node_inds, adj_mat_inds, init_hydrogens, init_charge,
           init_is_in_ring, init_is_aromatic, init_chirality,
           n_emb, e_emb, h_emb, charge_emb, ring_emb, arom_emb, chir_emb):
    eidx2 = (adj_mat_inds.reshape(-1, 1) * 2
             + jnp.arange(2, dtype=adj_mat_inds.dtype))
    eidx2 = eidx2.reshape(GRID, 1, RE)
    nidx = jnp.stack([a.reshape(GRID, RN) for a in
                      (node_inds, init_hydrogens, init_charge,
                       init_is_in_ring, init_is_aromatic, init_chirality)],
                     axis=1)

    e_tab = jnp.zeros((16, DIM_H), jnp.float32).at[:2 * NUM_EDGE_CLASSES].set(
        e_emb.reshape(NUM_EDGE_CLASSES, DIM_H, DIM_K)
             .transpose(0, 2, 1).reshape(2 * NUM_EDGE_CLASSES, DIM_H))
    n_tab = jnp.zeros((128, DIM_H), jnp.float32)
    for tab, off in zip((n_emb, h_emb, charge_emb, ring_emb, arom_emb, chir_emb),
                        _NODE_OFFSETS):
        n_tab = n_tab.at[off:off + tab.shape[0]].set(tab)

    edge_out, node_out = pl.pallas_call(
        _fused_body,
        grid=(GRID,),
        in_specs=[
            pl.BlockSpec((1, 1, RE), lambda i: (i, 0, 0)),
            pl.BlockSpec((1, 6, RN), lambda i: (i, 0, 0)),
            pl.BlockSpec((16, DIM_H), lambda i: (0, 0)),
            pl.BlockSpec((128, DIM_H), lambda i: (0, 0)),
        ],
        out_specs=[
            pl.BlockSpec((RE, DIM_H), lambda i: (i, 0)),
            pl.BlockSpec((RN, DIM_H), lambda i: (i, 0)),
        ],
        out_shape=[
            jax.ShapeDtypeStruct((B * N * N * DIM_K, DIM_H), jnp.float32),
            jax.ShapeDtypeStruct((B * N, DIM_H), jnp.float32),
        ],
    )(eidx2, nidx, e_tab, n_tab)

    edge5 = edge_out.reshape(B, N, N, DIM_K, DIM_H).swapaxes(-1, -2)
    return (node_out.reshape(B, N, DIM_H), edge5)


# speedup vs baseline: 36.9785x; 1.0235x over previous
import jax
import jax.numpy as jnp
from jax.experimental import pallas as pl
from jax.experimental.pallas import tpu as pltpu

B = 1024
N = 32
DIM_H = 128
DIM_K = 2
NUM_NODE_CLASSES = 42
NUM_EDGE_CLASSES = 6
NUM_H_CLASSES = 6
NUM_CHARGE_CLASSES = 6
NUM_RING_CLASSES = 3
NUM_AROM_CLASSES = 3
NUM_CHIR_CLASSES = 5

_NODE_SIZES = (NUM_NODE_CLASSES, NUM_H_CLASSES, NUM_CHARGE_CLASSES,
               NUM_RING_CLASSES, NUM_AROM_CLASSES, NUM_CHIR_CLASSES)
_NODE_OFFSETS = tuple(sum(_NODE_SIZES[:i]) for i in range(len(_NODE_SIZES)))

GRID = 64
RE = (B * N * N * DIM_K) // GRID
RN = (B * N) // GRID


def _fused_body(eidx_ref, nidx_ref, e_tab_ref, n_tab_ref,
                edge_out_ref, node_out_ref):
    idx8 = eidx_ref[0]
    eidx = jnp.concatenate(
        [jnp.broadcast_to(idx8[s:s + 1, :], (16, RE // 8)) for s in range(8)],
        axis=1)
    e_iota = jax.lax.broadcasted_iota(jnp.int32, (16, RE), 0)
    e_onehot_t = (eidx == e_iota).astype(jnp.float32)
    edge_out_ref[...] = jax.lax.dot_general(
        e_onehot_t, e_tab_ref[...], (((0,), (0,)), ((), ())),
        preferred_element_type=jnp.float32)

    n_iota = jax.lax.broadcasted_iota(jnp.int32, (RN, 128), 1)
    m = jnp.zeros((RN, 128), jnp.float32)
    for t, off in enumerate(_NODE_OFFSETS):
        idx_t = nidx_ref[0, t, :]
        m = m + (idx_t[:, None] + off == n_iota).astype(jnp.float32)
    node_out_ref[...] = jnp.dot(m, n_tab_ref[...],
                                preferred_element_type=jnp.float32)


def kernel(node_inds, adj_mat_inds, init_hydrogens, init_charge,
           init_is_in_ring, init_is_aromatic, init_chirality,
           n_emb, e_emb, h_emb, charge_emb, ring_emb, arom_emb, chir_emb):
    eidx2 = (adj_mat_inds.reshape(-1, 1) * 2
             + jnp.arange(2, dtype=adj_mat_inds.dtype))
    eidx2 = eidx2.reshape(GRID, 8, RE // 8)
    nidx = jnp.stack([a.reshape(GRID, RN) for a in
                      (node_inds, init_hydrogens, init_charge,
                       init_is_in_ring, init_is_aromatic, init_chirality)],
                     axis=1)

    e_tab = jnp.zeros((16, DIM_H), jnp.float32).at[:2 * NUM_EDGE_CLASSES].set(
        e_emb.reshape(NUM_EDGE_CLASSES, DIM_H, DIM_K)
             .transpose(0, 2, 1).reshape(2 * NUM_EDGE_CLASSES, DIM_H))
    n_tab = jnp.zeros((128, DIM_H), jnp.float32)
    for tab, off in zip((n_emb, h_emb, charge_emb, ring_emb, arom_emb, chir_emb),
                        _NODE_OFFSETS):
        n_tab = n_tab.at[off:off + tab.shape[0]].set(tab)

    edge_out, node_out = pl.pallas_call(
        _fused_body,
        grid=(GRID,),
        in_specs=[
            pl.BlockSpec((1, 8, RE // 8), lambda i: (i, 0, 0)),
            pl.BlockSpec((1, 6, RN), lambda i: (i, 0, 0)),
            pl.BlockSpec((16, DIM_H), lambda i: (0, 0)),
            pl.BlockSpec((128, DIM_H), lambda i: (0, 0)),
        ],
        out_specs=[
            pl.BlockSpec((RE, DIM_H), lambda i: (i, 0)),
            pl.BlockSpec((RN, DIM_H), lambda i: (i, 0)),
        ],
        out_shape=[
            jax.ShapeDtypeStruct((B * N * N * DIM_K, DIM_H), jnp.float32),
            jax.ShapeDtypeStruct((B * N, DIM_H), jnp.float32),
        ],
    )(eidx2, nidx, e_tab, n_tab)

    edge5 = edge_out.reshape(B, N, N, DIM_K, DIM_H).swapaxes(-1, -2)
    return (node_out.reshape(B, N, DIM_H), edge5)
